# baseline (speedup 1.0000x reference)
import functools

import jax
import jax.numpy as jnp
from jax.experimental import pallas as pl

N = 10000
E = 640000
IN_NF = 128
HID = 32
A_NF = 16
XP = 16
BE = 6400
BN = 2000


def _sigmoid(z):
    return 1.0 / (1.0 + jnp.exp(-z))


def _silu(z):
    return z * _sigmoid(z)


def _ln(z, g, b, eps=1e-5):
    mu = jnp.mean(z, axis=-1, keepdims=True)
    var = jnp.mean((z - mu) ** 2, axis=-1, keepdims=True)
    return (z - mu) * jax.lax.rsqrt(var + eps) * g + b


def _edge_kernel(hi_ref, hj_ref, xi_ref, xj_ref, ea_ref,
                 w1hi_ref, w1hj_ref, w1ea_ref, w1d2_ref, b1_ref, g1_ref, be1_ref,
                 w2_ref, b2_ref, g2_ref, be2_ref,
                 wx1_ref, bx1_ref, gx_ref, bex_ref, wx2_ref, bx2_ref,
                 mx_ref, mh_ref):
    hi = hi_ref[...]
    hj = hj_ref[...]
    xi = xi_ref[...]
    xj = xj_ref[...]
    diff = xi - xj
    d2 = jnp.sum(diff * diff, axis=-1, keepdims=True)
    z = (jnp.dot(hi, w1hi_ref[...], preferred_element_type=jnp.float32)
         + jnp.dot(hj, w1hj_ref[...], preferred_element_type=jnp.float32)
         + jnp.dot(ea_ref[...], w1ea_ref[...], preferred_element_type=jnp.float32)
         + d2 * w1d2_ref[...]
         + b1_ref[...])
    z = _silu(_ln(z, g1_ref[...], be1_ref[...]))
    z = jnp.dot(z, w2_ref[...], preferred_element_type=jnp.float32) + b2_ref[...]
    mh = _silu(_ln(z, g2_ref[...], be2_ref[...]))
    t = jnp.dot(mh, wx1_ref[...], preferred_element_type=jnp.float32) + bx1_ref[...]
    t = _silu(_ln(t, gx_ref[...], bex_ref[...]))
    px = jnp.dot(t, wx2_ref[...], preferred_element_type=jnp.float32) + bx2_ref[...]
    mx_ref[...] = diff * px
    mh_ref[...] = mh


def _edge_mlp(hi, hj, xi, xj, ea, p):
    w1 = p["e1"]["W"]
    ops = dict(
        w1hi=w1[:HID], w1hj=w1[HID:2 * HID], w1ea=w1[2 * HID + 1:],
        w1d2=w1[2 * HID:2 * HID + 1], b1=p["e1"]["b"][None, :],
        g1=p["e_ln1"]["g"][None, :], be1=p["e_ln1"]["b"][None, :],
        w2=p["e2"]["W"], b2=p["e2"]["b"][None, :],
        g2=p["e_ln2"]["g"][None, :], be2=p["e_ln2"]["b"][None, :],
        wx1=p["x1"]["W"], bx1=p["x1"]["b"][None, :],
        gx=p["x_ln"]["g"][None, :], bex=p["x_ln"]["b"][None, :],
        wx2=p["x2"]["W"], bx2=p["x2"]["b"][None, :],
    )
    grid = (E // BE,)
    eb = lambda f: pl.BlockSpec((BE, f), lambda i: (i, 0))
    full = lambda a: pl.BlockSpec(a.shape, lambda i: (0,) * a.ndim)
    return pl.pallas_call(
        _edge_kernel,
        grid=grid,
        in_specs=[eb(HID), eb(HID), eb(XP), eb(XP), eb(A_NF)]
                 + [full(v) for v in ops.values()],
        out_specs=[eb(XP), eb(HID)],
        out_shape=[jax.ShapeDtypeStruct((E, XP), jnp.float32),
                   jax.ShapeDtypeStruct((E, HID), jnp.float32)],
    )(hi, hj, xi, xj, ea, *ops.values())


def _node_kernel(h_ref, x_ref, mh_ref, mx_ref, c_ref,
                 wh1h_ref, wh1m_ref, bh1_ref, gh_ref, beh_ref,
                 wh2_ref, bh2_ref,
                 xo_ref, ho_ref):
    h = h_ref[...]
    z = (jnp.dot(h, wh1h_ref[...], preferred_element_type=jnp.float32)
         + jnp.dot(mh_ref[...], wh1m_ref[...], preferred_element_type=jnp.float32)
         + bh1_ref[...])
    z = _silu(_ln(z, gh_ref[...], beh_ref[...]))
    ho_ref[...] = (jnp.dot(z, wh2_ref[...], preferred_element_type=jnp.float32)
                   + bh2_ref[...] + h)
    xo_ref[...] = x_ref[...] + mx_ref[...] / c_ref[...]


def _node_mlp(h, x, mh_aggr, mx_aggr, c, p):
    wh1 = p["h1"]["W"]
    ops = dict(
        wh1h=wh1[:HID], wh1m=wh1[HID:], bh1=p["h1"]["b"][None, :],
        gh=p["h_ln"]["g"][None, :], beh=p["h_ln"]["b"][None, :],
        wh2=p["h2"]["W"], bh2=p["h2"]["b"][None, :],
    )
    grid = (N // BN,)
    nb = lambda f: pl.BlockSpec((BN, f), lambda i: (i, 0))
    full = lambda a: pl.BlockSpec(a.shape, lambda i: (0,) * a.ndim)
    xo, ho = pl.pallas_call(
        _node_kernel,
        grid=grid,
        in_specs=[nb(HID), nb(XP), nb(HID), nb(XP), nb(1)]
                 + [full(v) for v in ops.values()],
        out_specs=[nb(XP), nb(HID)],
        out_shape=[jax.ShapeDtypeStruct((N, XP), jnp.float32),
                   jax.ShapeDtypeStruct((N, HID), jnp.float32)],
    )(h, x, mh_aggr, mx_aggr, c, *ops.values())
    return xo, ho


def kernel(x, h, edges, edge_attr, params):
    src = edges[0]
    dst = edges[1]
    c = jax.ops.segment_sum(jnp.ones((E,), jnp.float32), src, num_segments=N)[:, None]
    xp = jnp.pad(x, ((0, 0), (0, XP - 3)))
    h = h @ params["emb"]["W"] + params["emb"]["b"]
    for p in params["layers"]:
        hi = h[dst]
        hj = h[src]
        xi = xp[dst]
        xj = xp[src]
        mx, mh = _edge_mlp(hi, hj, xi, xj, edge_attr, p)
        mx_aggr = jax.ops.segment_sum(mx, dst, num_segments=N)
        mh_aggr = jax.ops.segment_sum(mh, dst, num_segments=N)
        xp, h = _node_mlp(h, xp, mh_aggr, mx_aggr, c, p)
    h = h @ params["emb_out"]["W"] + params["emb_out"]["b"]
    return (xp[:, :3], h)


# speedup vs baseline: 2.6874x; 2.6874x over previous
import functools

import jax
import jax.numpy as jnp
from jax import lax
from jax.experimental import pallas as pl
from jax.experimental.pallas import tpu as pltpu
from jax.experimental.pallas import tpu_sc as plsc

N = 10000
E = 640000
IN_NF = 128
HID = 32
A_NF = 16
XP = 16
TF = HID + XP
BE = 6400
BN = 2000

NC = 2
NS = 16
NW = NC * NS
EPW = E // NW
GC = 800
GK = 80


def _gather_sc(table, dst, src):
    mesh = plsc.VectorSubcoreMesh(core_axis_name="c", subcore_axis_name="s")

    @functools.partial(
        pl.kernel, mesh=mesh,
        out_type=[jax.ShapeDtypeStruct((E, TF), jnp.float32),
                  jax.ShapeDtypeStruct((E, TF), jnp.float32)],
        scratch_types=[pltpu.VMEM((GC,), jnp.int32),
                       pltpu.VMEM((GC,), jnp.int32),
                       pltpu.VMEM((GC, TF), jnp.float32),
                       pltpu.VMEM((GC, TF), jnp.float32),
                       pltpu.SemaphoreType.DMA],
        compiler_params=pltpu.CompilerParams(use_tc_tiling_on_sc=False),
    )
    def k(table_hbm, dst_hbm, src_hbm, gi_hbm, gj_hbm, idxd_v, idxs_v, rd_v, rs_v, sem):
        wid = lax.axis_index("s") * NC + lax.axis_index("c")
        base0 = wid * EPW

        def body(t, carry):
            base = base0 + t * GC
            pltpu.sync_copy(dst_hbm.at[pl.ds(base, GC)], idxd_v)
            pltpu.sync_copy(src_hbm.at[pl.ds(base, GC)], idxs_v)
            cps = []
            for j in range(GC // GK):
                s = pl.ds(j * GK, GK)
                cps.append(pltpu.async_copy(table_hbm.at[idxd_v.at[s]],
                                            rd_v.at[s], sem))
                cps.append(pltpu.async_copy(table_hbm.at[idxs_v.at[s]],
                                            rs_v.at[s], sem))
            for cp in cps:
                cp.wait()
            pltpu.sync_copy(rd_v, gi_hbm.at[pl.ds(base, GC)])
            pltpu.sync_copy(rs_v, gj_hbm.at[pl.ds(base, GC)])
            return carry

        lax.fori_loop(0, EPW // GC, body, 0)

    return k(table, dst, src)


def _sigmoid(z):
    return 1.0 / (1.0 + jnp.exp(-z))


def _silu(z):
    return z * _sigmoid(z)


def _ln(z, g, b, eps=1e-5):
    mu = jnp.mean(z, axis=-1, keepdims=True)
    var = jnp.mean((z - mu) ** 2, axis=-1, keepdims=True)
    return (z - mu) * jax.lax.rsqrt(var + eps) * g + b


def _edge_kernel(gi_ref, gj_ref, ea_ref,
                 w1hi_ref, w1hj_ref, w1ea_ref, w1d2_ref, b1_ref, g1_ref, be1_ref,
                 w2_ref, b2_ref, g2_ref, be2_ref,
                 wx1_ref, bx1_ref, gx_ref, bex_ref, wx2_ref, bx2_ref,
                 mx_ref, mh_ref):
    gi = gi_ref[...]
    gj = gj_ref[...]
    hi = gi[:, :HID]
    hj = gj[:, :HID]
    xi = gi[:, HID:]
    xj = gj[:, HID:]
    diff = xi - xj
    d2 = jnp.sum(diff * diff, axis=-1, keepdims=True)
    z = (jnp.dot(hi, w1hi_ref[...], preferred_element_type=jnp.float32)
         + jnp.dot(hj, w1hj_ref[...], preferred_element_type=jnp.float32)
         + jnp.dot(ea_ref[...], w1ea_ref[...], preferred_element_type=jnp.float32)
         + d2 * w1d2_ref[...]
         + b1_ref[...])
    z = _silu(_ln(z, g1_ref[...], be1_ref[...]))
    z = jnp.dot(z, w2_ref[...], preferred_element_type=jnp.float32) + b2_ref[...]
    mh = _silu(_ln(z, g2_ref[...], be2_ref[...]))
    t = jnp.dot(mh, wx1_ref[...], preferred_element_type=jnp.float32) + bx1_ref[...]
    t = _silu(_ln(t, gx_ref[...], bex_ref[...]))
    px = jnp.dot(t, wx2_ref[...], preferred_element_type=jnp.float32) + bx2_ref[...]
    mx_ref[...] = diff * px
    mh_ref[...] = mh


def _edge_mlp(gi, gj, ea, p):
    w1 = p["e1"]["W"]
    ops = dict(
        w1hi=w1[:HID], w1hj=w1[HID:2 * HID], w1ea=w1[2 * HID + 1:],
        w1d2=w1[2 * HID:2 * HID + 1], b1=p["e1"]["b"][None, :],
        g1=p["e_ln1"]["g"][None, :], be1=p["e_ln1"]["b"][None, :],
        w2=p["e2"]["W"], b2=p["e2"]["b"][None, :],
        g2=p["e_ln2"]["g"][None, :], be2=p["e_ln2"]["b"][None, :],
        wx1=p["x1"]["W"], bx1=p["x1"]["b"][None, :],
        gx=p["x_ln"]["g"][None, :], bex=p["x_ln"]["b"][None, :],
        wx2=p["x2"]["W"], bx2=p["x2"]["b"][None, :],
    )
    grid = (E // BE,)
    eb = lambda f: pl.BlockSpec((BE, f), lambda i: (i, 0))
    full = lambda a: pl.BlockSpec(a.shape, lambda i: (0,) * a.ndim)
    return pl.pallas_call(
        _edge_kernel,
        grid=grid,
        in_specs=[eb(TF), eb(TF), eb(A_NF)]
                 + [full(v) for v in ops.values()],
        out_specs=[eb(XP), eb(HID)],
        out_shape=[jax.ShapeDtypeStruct((E, XP), jnp.float32),
                   jax.ShapeDtypeStruct((E, HID), jnp.float32)],
    )(gi, gj, ea, *ops.values())


def _node_kernel(h_ref, x_ref, mh_ref, mx_ref, c_ref,
                 wh1h_ref, wh1m_ref, bh1_ref, gh_ref, beh_ref,
                 wh2_ref, bh2_ref,
                 xo_ref, ho_ref):
    h = h_ref[...]
    z = (jnp.dot(h, wh1h_ref[...], preferred_element_type=jnp.float32)
         + jnp.dot(mh_ref[...], wh1m_ref[...], preferred_element_type=jnp.float32)
         + bh1_ref[...])
    z = _silu(_ln(z, gh_ref[...], beh_ref[...]))
    ho_ref[...] = (jnp.dot(z, wh2_ref[...], preferred_element_type=jnp.float32)
                   + bh2_ref[...] + h)
    xo_ref[...] = x_ref[...] + mx_ref[...] / c_ref[...]


def _node_mlp(h, x, mh_aggr, mx_aggr, c, p):
    wh1 = p["h1"]["W"]
    ops = dict(
        wh1h=wh1[:HID], wh1m=wh1[HID:], bh1=p["h1"]["b"][None, :],
        gh=p["h_ln"]["g"][None, :], beh=p["h_ln"]["b"][None, :],
        wh2=p["h2"]["W"], bh2=p["h2"]["b"][None, :],
    )
    grid = (N // BN,)
    nb = lambda f: pl.BlockSpec((BN, f), lambda i: (i, 0))
    full = lambda a: pl.BlockSpec(a.shape, lambda i: (0,) * a.ndim)
    xo, ho = pl.pallas_call(
        _node_kernel,
        grid=grid,
        in_specs=[nb(HID), nb(XP), nb(HID), nb(XP), nb(1)]
                 + [full(v) for v in ops.values()],
        out_specs=[nb(XP), nb(HID)],
        out_shape=[jax.ShapeDtypeStruct((N, XP), jnp.float32),
                   jax.ShapeDtypeStruct((N, HID), jnp.float32)],
    )(h, x, mh_aggr, mx_aggr, c, *ops.values())
    return xo, ho


def kernel(x, h, edges, edge_attr, params):
    src = edges[0]
    dst = edges[1]
    c = jax.ops.segment_sum(jnp.ones((E,), jnp.float32), src, num_segments=N)[:, None]
    xp = jnp.pad(x, ((0, 0), (0, XP - 3)))
    h = h @ params["emb"]["W"] + params["emb"]["b"]
    for p in params["layers"]:
        table = jnp.concatenate([h, xp], axis=1)
        gi, gj = _gather_sc(table, dst, src)
        mx, mh = _edge_mlp(gi, gj, edge_attr, p)
        mx_aggr = jax.ops.segment_sum(mx, dst, num_segments=N)
        mh_aggr = jax.ops.segment_sum(mh, dst, num_segments=N)
        xp, h = _node_mlp(h, xp, mh_aggr, mx_aggr, c, p)
    h = h @ params["emb_out"]["W"] + params["emb_out"]["b"]
    return (xp[:, :3], h)


# speedup vs baseline: 5.1556x; 1.9184x over previous
import functools

import jax
import jax.numpy as jnp
from jax import lax
from jax.experimental import pallas as pl
from jax.experimental.pallas import tpu as pltpu
from jax.experimental.pallas import tpu_sc as plsc

N = 10000
E = 640000
IN_NF = 128
HID = 32
A_NF = 16
XP = 16
TF = HID + XP
BE = 6400
BN = 2000

NC = 2
NS = 16
NW = NC * NS
EPW = E // NW
GC = 800
GK = 80


def _gather_sc(table, dst, src):
    mesh = plsc.VectorSubcoreMesh(core_axis_name="c", subcore_axis_name="s")

    @functools.partial(
        pl.kernel, mesh=mesh,
        out_type=[jax.ShapeDtypeStruct((E, TF), jnp.float32),
                  jax.ShapeDtypeStruct((E, TF), jnp.float32)],
        scratch_types=[pltpu.VMEM((GC,), jnp.int32),
                       pltpu.VMEM((GC,), jnp.int32),
                       pltpu.VMEM((GC, TF), jnp.float32),
                       pltpu.VMEM((GC, TF), jnp.float32),
                       pltpu.SemaphoreType.DMA],
        compiler_params=pltpu.CompilerParams(use_tc_tiling_on_sc=False),
    )
    def k(table_hbm, dst_hbm, src_hbm, gi_hbm, gj_hbm, idxd_v, idxs_v, rd_v, rs_v, sem):
        wid = lax.axis_index("s") * NC + lax.axis_index("c")
        base0 = wid * EPW

        def body(t, carry):
            base = base0 + t * GC
            pltpu.sync_copy(dst_hbm.at[pl.ds(base, GC)], idxd_v)
            pltpu.sync_copy(src_hbm.at[pl.ds(base, GC)], idxs_v)
            cps = []
            for j in range(GC // GK):
                s = pl.ds(j * GK, GK)
                cps.append(pltpu.async_copy(table_hbm.at[idxd_v.at[s]],
                                            rd_v.at[s], sem))
                cps.append(pltpu.async_copy(table_hbm.at[idxs_v.at[s]],
                                            rs_v.at[s], sem))
            for cp in cps:
                cp.wait()
            pltpu.sync_copy(rd_v, gi_hbm.at[pl.ds(base, GC)])
            pltpu.sync_copy(rs_v, gj_hbm.at[pl.ds(base, GC)])
            return carry

        lax.fori_loop(0, EPW // GC, body, 0)

    return k(table, dst, src)


def _sigmoid(z):
    return 1.0 / (1.0 + jnp.exp(-z))


def _silu(z):
    return z * _sigmoid(z)


def _ln(z, g, b, eps=1e-5):
    mu = jnp.mean(z, axis=-1, keepdims=True)
    var = jnp.mean((z - mu) ** 2, axis=-1, keepdims=True)
    return (z - mu) * jax.lax.rsqrt(var + eps) * g + b


SB = 80
SJ = 10
SCC = SB * SJ


def _scatter_sc(m, dst3d, zro):
    mesh = plsc.VectorSubcoreMesh(core_axis_name="c", subcore_axis_name="s")

    @functools.partial(
        pl.kernel, mesh=mesh,
        out_type=jax.ShapeDtypeStruct((NC, N, TF), jnp.float32),
        scratch_types=[pltpu.VMEM((SJ, SB), jnp.int32),
                       pltpu.VMEM((SCC, TF), jnp.float32),
                       pltpu.VMEM_SHARED((N, TF), jnp.float32)],
        compiler_params=pltpu.CompilerParams(use_tc_tiling_on_sc=False),
    )
    def k(m_hbm, dst_hbm, zro_hbm, out_hbm, idx_v, msg_v, shared):
        cid = lax.axis_index("c")
        sid = lax.axis_index("s")
        wid = sid * NC + cid

        @pl.when(sid < 10)
        def _():
            pltpu.sync_copy(zro_hbm, shared.at[pl.ds(sid * 1000, 1000)])

        plsc.subcore_barrier()

        def body(t, carry):
            blk = wid * (EPW // SCC) + t
            base = wid * EPW + t * SCC
            pltpu.sync_copy(dst_hbm.at[blk], idx_v)
            pltpu.sync_copy(m_hbm.at[pl.ds(base, SCC)], msg_v)
            for j in range(SJ):
                pltpu.sync_copy(msg_v.at[pl.ds(j * SB, SB)],
                                shared.at[idx_v.at[j]], add=True)
            return carry

        lax.fori_loop(0, EPW // SCC, body, 0)
        plsc.subcore_barrier()

        @pl.when(sid < 10)
        def _():
            pltpu.sync_copy(shared.at[pl.ds(sid * 1000, 1000)],
                            out_hbm.at[cid, pl.ds(sid * 1000, 1000)])

    return k(m, dst3d, zro)


def _degree_sc(src3d, ones, zro16):
    mesh = plsc.VectorSubcoreMesh(core_axis_name="c", subcore_axis_name="s")

    @functools.partial(
        pl.kernel, mesh=mesh,
        out_type=jax.ShapeDtypeStruct((NC, N, XP), jnp.float32),
        scratch_types=[pltpu.VMEM((SJ, SB), jnp.int32),
                       pltpu.VMEM((SB, XP), jnp.float32),
                       pltpu.VMEM_SHARED((N, XP), jnp.float32)],
        compiler_params=pltpu.CompilerParams(use_tc_tiling_on_sc=False),
    )
    def k(src_hbm, ones_hbm, zro_hbm, out_hbm, idx_v, ones_v, shared):
        cid = lax.axis_index("c")
        sid = lax.axis_index("s")
        wid = sid * NC + cid
        pltpu.sync_copy(ones_hbm, ones_v)

        @pl.when(sid < 10)
        def _():
            pltpu.sync_copy(zro_hbm, shared.at[pl.ds(sid * 1000, 1000)])

        plsc.subcore_barrier()

        def body(t, carry):
            blk = wid * (EPW // SCC) + t
            pltpu.sync_copy(src_hbm.at[blk], idx_v)
            for j in range(SJ):
                pltpu.sync_copy(ones_v, shared.at[idx_v.at[j]], add=True)
            return carry

        lax.fori_loop(0, EPW // SCC, body, 0)
        plsc.subcore_barrier()

        @pl.when(sid < 10)
        def _():
            pltpu.sync_copy(shared.at[pl.ds(sid * 1000, 1000)],
                            out_hbm.at[cid, pl.ds(sid * 1000, 1000)])

    return k(src3d, ones, zro16)


def _edge_kernel(gi_ref, gj_ref, ea_ref,
                 w1hi_ref, w1hj_ref, w1ea_ref, w1d2_ref, b1_ref, g1_ref, be1_ref,
                 w2_ref, b2_ref, g2_ref, be2_ref,
                 wx1_ref, bx1_ref, gx_ref, bex_ref, wx2_ref, bx2_ref,
                 m_ref):
    gi = gi_ref[...]
    gj = gj_ref[...]
    hi = gi[:, :HID]
    hj = gj[:, :HID]
    xi = gi[:, HID:]
    xj = gj[:, HID:]
    diff = xi - xj
    d2 = jnp.sum(diff * diff, axis=-1, keepdims=True)
    z = (jnp.dot(hi, w1hi_ref[...], preferred_element_type=jnp.float32)
         + jnp.dot(hj, w1hj_ref[...], preferred_element_type=jnp.float32)
         + jnp.dot(ea_ref[...], w1ea_ref[...], preferred_element_type=jnp.float32)
         + d2 * w1d2_ref[...]
         + b1_ref[...])
    z = _silu(_ln(z, g1_ref[...], be1_ref[...]))
    z = jnp.dot(z, w2_ref[...], preferred_element_type=jnp.float32) + b2_ref[...]
    mh = _silu(_ln(z, g2_ref[...], be2_ref[...]))
    t = jnp.dot(mh, wx1_ref[...], preferred_element_type=jnp.float32) + bx1_ref[...]
    t = _silu(_ln(t, gx_ref[...], bex_ref[...]))
    px = jnp.dot(t, wx2_ref[...], preferred_element_type=jnp.float32) + bx2_ref[...]
    m_ref[...] = jnp.concatenate([diff * px, mh], axis=-1)


def _edge_mlp(gi, gj, ea, p):
    w1 = p["e1"]["W"]
    ops = dict(
        w1hi=w1[:HID], w1hj=w1[HID:2 * HID], w1ea=w1[2 * HID + 1:],
        w1d2=w1[2 * HID:2 * HID + 1], b1=p["e1"]["b"][None, :],
        g1=p["e_ln1"]["g"][None, :], be1=p["e_ln1"]["b"][None, :],
        w2=p["e2"]["W"], b2=p["e2"]["b"][None, :],
        g2=p["e_ln2"]["g"][None, :], be2=p["e_ln2"]["b"][None, :],
        wx1=p["x1"]["W"], bx1=p["x1"]["b"][None, :],
        gx=p["x_ln"]["g"][None, :], bex=p["x_ln"]["b"][None, :],
        wx2=p["x2"]["W"], bx2=p["x2"]["b"][None, :],
    )
    grid = (E // BE,)
    eb = lambda f: pl.BlockSpec((BE, f), lambda i: (i, 0))
    full = lambda a: pl.BlockSpec(a.shape, lambda i: (0,) * a.ndim)
    return pl.pallas_call(
        _edge_kernel,
        grid=grid,
        in_specs=[eb(TF), eb(TF), eb(A_NF)]
                 + [full(v) for v in ops.values()],
        out_specs=eb(TF),
        out_shape=jax.ShapeDtypeStruct((E, TF), jnp.float32),
    )(gi, gj, ea, *ops.values())


def _node_kernel(h_ref, x_ref, a0_ref, a1_ref, c_ref,
                 wh1h_ref, wh1m_ref, bh1_ref, gh_ref, beh_ref,
                 wh2_ref, bh2_ref,
                 xo_ref, ho_ref):
    h = h_ref[...]
    aggr = a0_ref[...] + a1_ref[...]
    mx_a = aggr[:, :XP]
    mh_a = aggr[:, XP:]
    z = (jnp.dot(h, wh1h_ref[...], preferred_element_type=jnp.float32)
         + jnp.dot(mh_a, wh1m_ref[...], preferred_element_type=jnp.float32)
         + bh1_ref[...])
    z = _silu(_ln(z, gh_ref[...], beh_ref[...]))
    ho_ref[...] = (jnp.dot(z, wh2_ref[...], preferred_element_type=jnp.float32)
                   + bh2_ref[...] + h)
    xo_ref[...] = x_ref[...] + mx_a / c_ref[...]


def _node_mlp(h, x, a0, a1, c, p):
    wh1 = p["h1"]["W"]
    ops = dict(
        wh1h=wh1[:HID], wh1m=wh1[HID:], bh1=p["h1"]["b"][None, :],
        gh=p["h_ln"]["g"][None, :], beh=p["h_ln"]["b"][None, :],
        wh2=p["h2"]["W"], bh2=p["h2"]["b"][None, :],
    )
    grid = (N // BN,)
    nb = lambda f: pl.BlockSpec((BN, f), lambda i: (i, 0))
    full = lambda a: pl.BlockSpec(a.shape, lambda i: (0,) * a.ndim)
    xo, ho = pl.pallas_call(
        _node_kernel,
        grid=grid,
        in_specs=[nb(HID), nb(XP), nb(TF), nb(TF), nb(1)]
                 + [full(v) for v in ops.values()],
        out_specs=[nb(XP), nb(HID)],
        out_shape=[jax.ShapeDtypeStruct((N, XP), jnp.float32),
                   jax.ShapeDtypeStruct((N, HID), jnp.float32)],
    )(h, x, a0, a1, c, *ops.values())
    return xo, ho


def kernel(x, h, edges, edge_attr, params):
    src = edges[0]
    dst = edges[1]
    src3d = src.reshape(E // SCC, SJ, SB)
    dst3d = dst.reshape(E // SCC, SJ, SB)
    ones = jnp.ones((SB, XP), jnp.float32)
    zro16 = jnp.zeros((1000, XP), jnp.float32)
    zro = jnp.zeros((1000, TF), jnp.float32)
    cp = _degree_sc(src3d, ones, zro16)
    c = (cp[0, :, :1] + cp[1, :, :1])
    xp = jnp.pad(x, ((0, 0), (0, XP - 3)))
    h = h @ params["emb"]["W"] + params["emb"]["b"]
    for p in params["layers"]:
        table = jnp.concatenate([h, xp], axis=1)
        gi, gj = _gather_sc(table, dst, src)
        m = _edge_mlp(gi, gj, edge_attr, p)
        aggr = _scatter_sc(m, dst3d, zro)
        xp, h = _node_mlp(h, xp, aggr[0], aggr[1], c, p)
    h = h @ params["emb_out"]["W"] + params["emb_out"]["b"]
    return (xp[:, :3], h)
